# baseline (speedup 1.0000x reference)
import functools

import jax
import jax.numpy as jnp
from jax import lax
from jax.experimental import pallas as pl
from jax.experimental.pallas import tpu as pltpu
from jax.experimental.pallas import tpu_sc as plsc

N = 10000
E = 320000
D_IN = 128
D_H = 128
D_OUT = 64
NEG = 0.2

NC, NS, L = 2, 16, 16
NW = NC * NS
K = 128
EROWS = E // K
NBK = 80
EROWS_P = NW * NBK
SP = 80
RPT = 624
_ZOFF = (0, 80, 160, 240, 320, 400, 480, 560)

_MESH = plsc.VectorSubcoreMesh(
    core_axis_name="c", subcore_axis_name="s", num_cores=NC, num_subcores=NS)
_SC_PARAMS = pltpu.CompilerParams(
    use_tc_tiling_on_sc=False, needs_layout_passes=False)


def _nrows(wid):
    return jnp.minimum(EROWS - NBK * wid, NBK)


def _zero_rows(ref, nrows, ncol16):
    z = jnp.zeros((L,), jnp.float32)

    def body(j, carry):
        for cc in range(ncol16):
            ref[j, pl.ds(cc * L, L)] = z
        return carry

    lax.fori_loop(0, nrows, body, 0)


def _prep_body(x_ref, w_ref, al_ref, ar_ref, feat_ref, el_ref, er_ref, cv_ref,
               m_ref):
    i = pl.program_id(0)
    f = jnp.dot(x_ref[...], w_ref[...], preferred_element_type=jnp.float32)
    feat_ref[...] = f
    el = jnp.dot(f, al_ref[...], preferred_element_type=jnp.float32)
    er = jnp.dot(f, ar_ref[...], preferred_element_type=jnp.float32)
    el_ref[...] = el
    er_ref[...] = er

    @pl.when(i == 0)
    def _():
        m_ref[0] = jnp.float32(-1e30)
        m_ref[1] = jnp.float32(-1e30)

    m_ref[0] = jnp.maximum(m_ref[0], jnp.max(el))
    m_ref[1] = jnp.maximum(m_ref[1], jnp.max(er))
    z = m_ref[0] + m_ref[1]
    c = jnp.where(z >= 0, z, NEG * z)
    cv_ref[...] = jnp.full((8, 128), c, jnp.float32)


_PB = 2000


def _prep(x, w, al, ar):
    return pl.pallas_call(
        _prep_body,
        grid=(N // _PB,),
        in_specs=[
            pl.BlockSpec((_PB, D_IN), lambda i: (i, 0)),
            pl.BlockSpec((D_IN, D_H), lambda i: (0, 0)),
            pl.BlockSpec((D_H, 1), lambda i: (0, 0)),
            pl.BlockSpec((D_H, 1), lambda i: (0, 0)),
        ],
        out_specs=[
            pl.BlockSpec((_PB, D_H), lambda i: (i, 0)),
            pl.BlockSpec((_PB, 1), lambda i: (i, 0)),
            pl.BlockSpec((_PB, 1), lambda i: (i, 0)),
            pl.BlockSpec((8, 128), lambda i: (0, 0)),
        ],
        out_shape=[
            jax.ShapeDtypeStruct((N, D_H), jnp.float32),
            jax.ShapeDtypeStruct((N, 1), jnp.float32),
            jax.ShapeDtypeStruct((N, 1), jnp.float32),
            jax.ShapeDtypeStruct((8, 128), jnp.float32),
        ],
        scratch_shapes=[pltpu.SMEM((2,), jnp.float32)],
    )(x, w, al, ar)


def _edge_body(el_h, er_h, src_h, dst_h, cv_h, ex_h, s_h,
               el_v, er_v, src_v, dst_v, ex_v, s_v, cv_v, idx_v, s_sh):
    c = lax.axis_index("c")
    sid = lax.axis_index("s")
    wid = c * NS + sid
    rb = wid * NBK
    pltpu.sync_copy(el_h, el_v)
    pltpu.sync_copy(er_h, er_v)
    pltpu.sync_copy(src_h.at[pl.ds(rb, NBK)], src_v)
    pltpu.sync_copy(dst_h.at[pl.ds(rb, NBK)], dst_v)
    pltpu.sync_copy(cv_h.at[0], cv_v)

    _zero_rows(s_v, SP, 8)

    def ibody(j, carry):
        idx_v[pl.ds(j * L, L)] = lax.iota(jnp.int32, L) + j * L
        return carry

    lax.fori_loop(0, SP // L, ibody, 0)

    @pl.when(sid == 0)
    def _():
        pltpu.sync_copy(s_v, s_sh)

    plsc.subcore_barrier()

    cv = cv_v[pl.ds(0, L)]

    def ebody(r, carry):
        for g in range(K // L):
            sl = pl.ds(g * L, L)
            sv = src_v[r, sl]
            dv = dst_v[r, sl]
            elg = plsc.load_gather(el_v, [sv])
            erg = plsc.load_gather(er_v, [dv])
            z = elg + erg
            e = jnp.where(z >= 0, z, NEG * z)
            ex = jnp.exp(e - cv)
            ex_v[r, sl] = ex
            hi = lax.shift_right_logical(dv, 7)
            lo = lax.bitwise_and(dv, 127)
            plsc.addupdate_scatter(s_v, [hi, lo], ex)
        return carry

    nr = _nrows(wid)
    lax.fori_loop(0, nr, ebody, 0)
    zz = jnp.zeros((L,), jnp.float32)

    def zpad(r, carry):
        for g in range(K // L):
            ex_v[r, pl.ds(g * L, L)] = zz
        return carry

    lax.fori_loop(nr, NBK, zpad, 0)
    pltpu.sync_copy(ex_v, ex_h.at[pl.ds(rb, NBK)])
    pltpu.sync_copy(s_v, s_sh.at[idx_v], add=True)
    plsc.subcore_barrier()

    @pl.when(sid < SP // 8)
    def _():
        pltpu.sync_copy(s_sh.at[pl.ds(sid * 8, 8), :],
                        s_h.at[c, pl.ds(sid * 8, 8), :])


_edge = pl.kernel(
    _edge_body,
    out_type=[
        jax.ShapeDtypeStruct((EROWS_P, K), jnp.float32),
        jax.ShapeDtypeStruct((NC, SP, 128), jnp.float32),
    ],
    mesh=_MESH,
    compiler_params=_SC_PARAMS,
    scratch_types=[
        pltpu.VMEM((N,), jnp.float32),
        pltpu.VMEM((N,), jnp.float32),
        pltpu.VMEM((NBK, K), jnp.int32),
        pltpu.VMEM((NBK, K), jnp.int32),
        pltpu.VMEM((NBK, K), jnp.float32),
        pltpu.VMEM((SP, 128), jnp.float32),
        pltpu.VMEM((128,), jnp.float32),
        pltpu.VMEM((SP,), jnp.int32),
        pltpu.VMEM_SHARED((SP, 128), jnp.float32),
    ],
)


def _copy_out(acc_sh, agg_h, c, sid):
    rowbase = sid * RPT
    pltpu.sync_copy(acc_sh.at[pl.ds(rowbase, RPT), :],
                    agg_h.at[c, pl.ds(rowbase, RPT), :])

    @pl.when(sid == NS - 1)
    def _():
        pltpu.sync_copy(acc_sh.at[pl.ds(NS * RPT, N - NS * RPT), :],
                        agg_h.at[c, pl.ds(NS * RPT, N - NS * RPT), :])


def _zero_acc(rows_v, acc_sh, sid, ncol16):
    _zero_rows(rows_v, NBK, ncol16)
    rowbase = sid * RPT
    for off in _ZOFF:
        pltpu.sync_copy(rows_v.at[pl.ds(0, NBK), :],
                        acc_sh.at[pl.ds(rowbase + off, NBK), :])


CH = 40


def _scale_rows(al2_v, b, rv, d):
    def sg(g, carry):
        av16 = al2_v[b, pl.ds(g * L, L)]
        for j in range(L):
            av = jnp.broadcast_to(av16[j], (L,))
            r = g * L + j
            for cc in range(d // L):
                sl = pl.ds(cc * L, L)
                rv[r, sl] = rv[r, sl] * av
        return carry

    lax.fori_loop(0, K // L, sg, 0)


def _pipeline_chunk(tbl_h, acc_sh, src2_v, dst2_v, al2_v, r0, d):
    def batch(b, carry):
        pltpu.sync_copy(tbl_h.at[src2_v.at[b]], r0)
        _scale_rows(al2_v, b, r0, d)
        pltpu.sync_copy(r0, acc_sh.at[dst2_v.at[b]], add=True)
        return carry

    lax.fori_loop(0, CH, batch, 0)


def _spmm1_body(feat_h, src2_h, dst2_h, ex2_h, s_h, agg_h, al2_h,
                src2_v, dst2_v, al2_v, s_v, r0, acc_sh):
    c = lax.axis_index("c")
    sid = lax.axis_index("s")
    wid = c * NS + sid
    rb = wid * NBK
    pltpu.sync_copy(s_h, s_v)

    _zero_acc(r0, acc_sh, sid, D_H // L)
    plsc.subcore_barrier()

    def chunk(t, carry):
        cb = rb + t * CH
        pltpu.sync_copy(src2_h.at[pl.ds(cb, CH)], src2_v)
        pltpu.sync_copy(dst2_h.at[pl.ds(cb, CH)], dst2_v)
        pltpu.sync_copy(ex2_h.at[pl.ds(cb, CH)], al2_v)

        def albody(b, carry2):
            for g in range(K // L):
                sl = pl.ds(g * L, L)
                dv = dst2_v[b, sl]
                hi = lax.shift_right_logical(dv, 7)
                lo = lax.bitwise_and(dv, 127)
                sg = plsc.load_gather(s_v, [hi, lo])
                al2_v[b, sl] = al2_v[b, sl] / (sg + 1e-9)
            return carry2

        lax.fori_loop(0, CH, albody, 0)
        pltpu.sync_copy(al2_v, al2_h.at[pl.ds(cb, CH)])
        _pipeline_chunk(feat_h, acc_sh, src2_v, dst2_v, al2_v, r0, D_H)
        return carry

    lax.fori_loop(0, NBK // CH, chunk, 0)
    plsc.subcore_barrier()
    _copy_out(acc_sh, agg_h, c, sid)


_spmm1 = pl.kernel(
    _spmm1_body,
    out_type=[
        jax.ShapeDtypeStruct((NC, N, D_H), jnp.float32),
        jax.ShapeDtypeStruct((EROWS_P, K), jnp.float32),
    ],
    mesh=_MESH,
    compiler_params=_SC_PARAMS,
    scratch_types=[
        pltpu.VMEM((CH, K), jnp.int32),
        pltpu.VMEM((CH, K), jnp.int32),
        pltpu.VMEM((CH, K), jnp.float32),
        pltpu.VMEM((SP, 128), jnp.float32),
        pltpu.VMEM((K, D_H), jnp.float32),
        pltpu.VMEM_SHARED((N, D_H), jnp.float32),
    ],
)


def _spmm_body(d, tbl_h, src2_h, dst2_h, al2_h, agg_h,
               src2_v, dst2_v, al2_v, r0, acc_sh):
    c = lax.axis_index("c")
    sid = lax.axis_index("s")
    wid = c * NS + sid
    rb = wid * NBK

    _zero_acc(r0, acc_sh, sid, d // L)
    plsc.subcore_barrier()

    def chunk(t, carry):
        cb = rb + t * CH
        pltpu.sync_copy(src2_h.at[pl.ds(cb, CH)], src2_v)
        pltpu.sync_copy(dst2_h.at[pl.ds(cb, CH)], dst2_v)
        pltpu.sync_copy(al2_h.at[pl.ds(cb, CH)], al2_v)
        _pipeline_chunk(tbl_h, acc_sh, src2_v, dst2_v, al2_v, r0, d)
        return carry

    lax.fori_loop(0, NBK // CH, chunk, 0)
    plsc.subcore_barrier()
    _copy_out(acc_sh, agg_h, c, sid)


def _make_spmm(d):
    return pl.kernel(
        functools.partial(_spmm_body, d),
        out_type=jax.ShapeDtypeStruct((NC, N, d), jnp.float32),
        mesh=_MESH,
        compiler_params=_SC_PARAMS,
        scratch_types=[
            pltpu.VMEM((CH, K), jnp.int32),
            pltpu.VMEM((CH, K), jnp.int32),
            pltpu.VMEM((CH, K), jnp.float32),
            pltpu.VMEM((K, d), jnp.float32),
            pltpu.VMEM_SHARED((N, d), jnp.float32),
        ],
    )


_spmm2 = _make_spmm(D_H)
_spmm3 = _make_spmm(D_OUT)


def _scomb_body(s2_ref, s_ref):
    s_ref[...] = s2_ref[0] + s2_ref[1]


def _scomb(s2):
    return pl.pallas_call(
        _scomb_body,
        grid=(1,),
        in_specs=[pl.BlockSpec((NC, SP, 128), lambda i: (0, 0, 0))],
        out_specs=pl.BlockSpec((SP, 128), lambda i: (0, 0)),
        out_shape=jax.ShapeDtypeStruct((SP, 128), jnp.float32),
    )(s2)


def _mid1_body(agg_ref, b_ref, w_ref, g_ref):
    h = jnp.maximum(agg_ref[0] + agg_ref[1] + b_ref[...], 0.0)
    g_ref[...] = jnp.dot(h, w_ref[...], preferred_element_type=jnp.float32)


def _mid1(agg, b, w):
    return pl.pallas_call(
        _mid1_body,
        grid=(N // _PB,),
        in_specs=[
            pl.BlockSpec((NC, _PB, D_H), lambda i: (0, i, 0)),
            pl.BlockSpec((1, D_H), lambda i: (0, 0)),
            pl.BlockSpec((D_H, D_H), lambda i: (0, 0)),
        ],
        out_specs=pl.BlockSpec((_PB, D_H), lambda i: (i, 0)),
        out_shape=jax.ShapeDtypeStruct((N, D_H), jnp.float32),
    )(agg, b, w)


def _mid2_body(agg_ref, b_ref, w_ref, q_ref):
    h = agg_ref[0] + agg_ref[1] + b_ref[...]
    q_ref[...] = jnp.dot(h, w_ref[...], preferred_element_type=jnp.float32)


def _mid2(agg, b, w):
    return pl.pallas_call(
        _mid2_body,
        grid=(N // _PB,),
        in_specs=[
            pl.BlockSpec((NC, _PB, D_H), lambda i: (0, i, 0)),
            pl.BlockSpec((1, D_H), lambda i: (0, 0)),
            pl.BlockSpec((D_H, D_OUT), lambda i: (0, 0)),
        ],
        out_specs=pl.BlockSpec((_PB, D_OUT), lambda i: (i, 0)),
        out_shape=jax.ShapeDtypeStruct((N, D_OUT), jnp.float32),
    )(agg, b, w)


def _final_body(agg_ref, b_ref, out_ref):
    out_ref[...] = agg_ref[0] + agg_ref[1] + b_ref[...]


def _final(agg, b):
    return pl.pallas_call(
        _final_body,
        grid=(N // _PB,),
        in_specs=[
            pl.BlockSpec((NC, _PB, D_OUT), lambda i: (0, i, 0)),
            pl.BlockSpec((1, D_OUT), lambda i: (0, 0)),
        ],
        out_specs=pl.BlockSpec((_PB, D_OUT), lambda i: (i, 0)),
        out_shape=jax.ShapeDtypeStruct((N, D_OUT), jnp.float32),
    )(agg, b)


def kernel(inputs, edge_index, W_gat, attn_l, attn_r, b_gat, W1, b1, W2, b2):
    src = edge_index[0]
    dst = edge_index[1]
    pad = ((0, EROWS_P - EROWS), (0, 0))
    src2 = jnp.pad(src.reshape(EROWS, K), pad)
    dst2 = jnp.pad(dst.reshape(EROWS, K), pad)
    feat, el2, er2, cvec = _prep(inputs, W_gat,
                                 attn_l.reshape(D_H, 1), attn_r.reshape(D_H, 1))
    el = el2.reshape(N)
    er = er2.reshape(N)
    ex2, s2 = _edge(el, er, src2, dst2, cvec)
    s = _scomb(s2)
    agg1, alpha2 = _spmm1(feat, src2, dst2, ex2, s)
    g = _mid1(agg1, b_gat.reshape(1, D_H), W1)
    agg2 = _spmm2(g, src2, dst2, alpha2)
    q = _mid2(agg2, b1.reshape(1, D_H), W2)
    agg3 = _spmm3(q, src2, dst2, alpha2)
    return _final(agg3, b2.reshape(1, D_OUT))


# speedup vs baseline: 1.9244x; 1.9244x over previous
import functools

import jax
import jax.numpy as jnp
from jax import lax
from jax.experimental import pallas as pl
from jax.experimental.pallas import tpu as pltpu
from jax.experimental.pallas import tpu_sc as plsc

N = 10000
E = 320000
D_IN = 128
D_H = 128
D_OUT = 64
NEG = 0.2

NC, NS, L = 2, 16, 16
NW = NC * NS
K = 128
EROWS = E // K
NBK = 80
EROWS_P = NW * NBK
SP = 80
RPT = 624
_ZOFF = (0, 80, 160, 240, 320, 400, 480, 560)

_MESH = plsc.VectorSubcoreMesh(
    core_axis_name="c", subcore_axis_name="s", num_cores=NC, num_subcores=NS)
_SC_PARAMS = pltpu.CompilerParams(
    use_tc_tiling_on_sc=False, needs_layout_passes=False)


def _nrows(wid):
    return jnp.minimum(EROWS - NBK * wid, NBK)


def _zero_rows(ref, nrows, ncol16):
    z = jnp.zeros((L,), jnp.float32)

    def body(j, carry):
        for cc in range(ncol16):
            ref[j, pl.ds(cc * L, L)] = z
        return carry

    lax.fori_loop(0, nrows, body, 0)


def _prep_body(x_ref, w_ref, al_ref, ar_ref, feat_ref, el_ref, er_ref, cv_ref,
               m_ref):
    i = pl.program_id(0)
    f = jnp.dot(x_ref[...], w_ref[...], preferred_element_type=jnp.float32)
    feat_ref[...] = f
    el = jnp.dot(f, al_ref[...], preferred_element_type=jnp.float32)
    er = jnp.dot(f, ar_ref[...], preferred_element_type=jnp.float32)
    el_ref[...] = el
    er_ref[...] = er

    @pl.when(i == 0)
    def _():
        m_ref[0] = jnp.float32(-1e30)
        m_ref[1] = jnp.float32(-1e30)

    m_ref[0] = jnp.maximum(m_ref[0], jnp.max(el))
    m_ref[1] = jnp.maximum(m_ref[1], jnp.max(er))
    z = m_ref[0] + m_ref[1]
    c = jnp.where(z >= 0, z, NEG * z)
    cv_ref[...] = jnp.full((8, 128), c, jnp.float32)


_PB = 2000


def _prep(x, w, al, ar):
    return pl.pallas_call(
        _prep_body,
        grid=(N // _PB,),
        in_specs=[
            pl.BlockSpec((_PB, D_IN), lambda i: (i, 0)),
            pl.BlockSpec((D_IN, D_H), lambda i: (0, 0)),
            pl.BlockSpec((D_H, 1), lambda i: (0, 0)),
            pl.BlockSpec((D_H, 1), lambda i: (0, 0)),
        ],
        out_specs=[
            pl.BlockSpec((_PB, D_H), lambda i: (i, 0)),
            pl.BlockSpec((_PB, 1), lambda i: (i, 0)),
            pl.BlockSpec((_PB, 1), lambda i: (i, 0)),
            pl.BlockSpec((8, 128), lambda i: (0, 0)),
        ],
        out_shape=[
            jax.ShapeDtypeStruct((N, D_H), jnp.float32),
            jax.ShapeDtypeStruct((N, 1), jnp.float32),
            jax.ShapeDtypeStruct((N, 1), jnp.float32),
            jax.ShapeDtypeStruct((8, 128), jnp.float32),
        ],
        scratch_shapes=[pltpu.SMEM((2,), jnp.float32)],
    )(x, w, al, ar)


def _edge_body(el_h, er_h, src_h, dst_h, cv_h, ex_h, s_h,
               el_v, er_v, src_v, dst_v, ex_v, s_v, cv_v, idx_v, s_sh):
    c = lax.axis_index("c")
    sid = lax.axis_index("s")
    wid = c * NS + sid
    rb = wid * NBK
    pltpu.sync_copy(el_h, el_v)
    pltpu.sync_copy(er_h, er_v)
    pltpu.sync_copy(src_h.at[pl.ds(rb, NBK)], src_v)
    pltpu.sync_copy(dst_h.at[pl.ds(rb, NBK)], dst_v)
    pltpu.sync_copy(cv_h.at[0], cv_v)

    _zero_rows(s_v, SP, 8)

    def ibody(j, carry):
        idx_v[pl.ds(j * L, L)] = lax.iota(jnp.int32, L) + j * L
        return carry

    lax.fori_loop(0, SP // L, ibody, 0)

    @pl.when(sid == 0)
    def _():
        pltpu.sync_copy(s_v, s_sh)

    plsc.subcore_barrier()

    cv = cv_v[pl.ds(0, L)]

    def ebody(r, carry):
        for g in range(K // L):
            sl = pl.ds(g * L, L)
            sv = src_v[r, sl]
            dv = dst_v[r, sl]
            elg = plsc.load_gather(el_v, [sv])
            erg = plsc.load_gather(er_v, [dv])
            z = elg + erg
            e = jnp.where(z >= 0, z, NEG * z)
            ex = jnp.exp(e - cv)
            ex_v[r, sl] = ex
            hi = lax.shift_right_logical(dv, 7)
            lo = lax.bitwise_and(dv, 127)
            plsc.addupdate_scatter(s_v, [hi, lo], ex)
        return carry

    nr = _nrows(wid)
    lax.fori_loop(0, nr, ebody, 0)
    zz = jnp.zeros((L,), jnp.float32)

    def zpad(r, carry):
        for g in range(K // L):
            ex_v[r, pl.ds(g * L, L)] = zz
        return carry

    lax.fori_loop(nr, NBK, zpad, 0)
    pltpu.sync_copy(ex_v, ex_h.at[pl.ds(rb, NBK)])
    pltpu.sync_copy(s_v, s_sh.at[idx_v], add=True)
    plsc.subcore_barrier()

    @pl.when(sid < SP // 8)
    def _():
        pltpu.sync_copy(s_sh.at[pl.ds(sid * 8, 8), :],
                        s_h.at[c, pl.ds(sid * 8, 8), :])


_edge = pl.kernel(
    _edge_body,
    out_type=[
        jax.ShapeDtypeStruct((EROWS_P, K), jnp.float32),
        jax.ShapeDtypeStruct((NC, SP, 128), jnp.float32),
    ],
    mesh=_MESH,
    compiler_params=_SC_PARAMS,
    scratch_types=[
        pltpu.VMEM((N,), jnp.float32),
        pltpu.VMEM((N,), jnp.float32),
        pltpu.VMEM((NBK, K), jnp.int32),
        pltpu.VMEM((NBK, K), jnp.int32),
        pltpu.VMEM((NBK, K), jnp.float32),
        pltpu.VMEM((SP, 128), jnp.float32),
        pltpu.VMEM((128,), jnp.float32),
        pltpu.VMEM((SP,), jnp.int32),
        pltpu.VMEM_SHARED((SP, 128), jnp.float32),
    ],
)


def _copy_out(acc_sh, agg_h, c, sid):
    rowbase = sid * RPT
    pltpu.sync_copy(acc_sh.at[pl.ds(rowbase, RPT), :],
                    agg_h.at[c, pl.ds(rowbase, RPT), :])

    @pl.when(sid == NS - 1)
    def _():
        pltpu.sync_copy(acc_sh.at[pl.ds(NS * RPT, N - NS * RPT), :],
                        agg_h.at[c, pl.ds(NS * RPT, N - NS * RPT), :])


def _zero_acc(rows_v, acc_sh, sid, ncol16):
    _zero_rows(rows_v, NBK, ncol16)
    rowbase = sid * RPT
    for off in _ZOFF:
        pltpu.sync_copy(rows_v.at[pl.ds(0, NBK), :],
                        acc_sh.at[pl.ds(rowbase + off, NBK), :])


CH = 16


def _scale_rows(al2_v, b, rv, d):
    def sg(g, carry):
        av16 = al2_v[b, pl.ds(g * L, L)]
        for j in range(L):
            av = jnp.broadcast_to(av16[j], (L,))
            r = g * L + j
            for cc in range(d // L):
                sl = pl.ds(cc * L, L)
                rv[r, sl] = rv[r, sl] * av
        return carry

    lax.fori_loop(0, K // L, sg, 0)


def _pipeline_chunk(tbl_h, acc_sh, src2_v, dst2_v, al2_v, r0, d, nrem):
    def batch(b, carry):
        pltpu.sync_copy(tbl_h.at[src2_v.at[b]], r0)
        _scale_rows(al2_v, b, r0, d)
        pltpu.sync_copy(r0, acc_sh.at[dst2_v.at[b]], add=True)
        return carry

    lax.fori_loop(0, nrem, batch, 0)


def _spmm1_body(feat_h, src2_h, dst2_h, ex2_h, s_h, agg_h, al2_h,
                src2_v, dst2_v, al2_v, s_v, r0, acc_sh):
    c = lax.axis_index("c")
    sid = lax.axis_index("s")
    wid = c * NS + sid
    rb = wid * NBK
    pltpu.sync_copy(s_h, s_v)
    nrows = _nrows(wid)

    _zero_acc(r0, acc_sh, sid, D_H // L)
    plsc.subcore_barrier()

    def chunk(t, carry):
        cb = rb + t * CH
        pltpu.sync_copy(src2_h.at[pl.ds(cb, CH)], src2_v)
        pltpu.sync_copy(dst2_h.at[pl.ds(cb, CH)], dst2_v)
        pltpu.sync_copy(ex2_h.at[pl.ds(cb, CH)], al2_v)
        nrem = jnp.clip(nrows - t * CH, 0, CH)

        def albody(b, carry2):
            for g in range(K // L):
                sl = pl.ds(g * L, L)
                dv = dst2_v[b, sl]
                hi = lax.shift_right_logical(dv, 7)
                lo = lax.bitwise_and(dv, 127)
                sg = plsc.load_gather(s_v, [hi, lo])
                al2_v[b, sl] = al2_v[b, sl] / (sg + 1e-9)
            return carry2

        lax.fori_loop(0, nrem, albody, 0)
        pltpu.sync_copy(al2_v, al2_h.at[pl.ds(cb, CH)])
        _pipeline_chunk(feat_h, acc_sh, src2_v, dst2_v, al2_v, r0, D_H, nrem)
        return carry

    lax.fori_loop(0, NBK // CH, chunk, 0)
    plsc.subcore_barrier()
    _copy_out(acc_sh, agg_h, c, sid)


_spmm1 = pl.kernel(
    _spmm1_body,
    out_type=[
        jax.ShapeDtypeStruct((NC, N, D_H), jnp.float32),
        jax.ShapeDtypeStruct((EROWS_P, K), jnp.float32),
    ],
    mesh=_MESH,
    compiler_params=_SC_PARAMS,
    scratch_types=[
        pltpu.VMEM((CH, K), jnp.int32),
        pltpu.VMEM((CH, K), jnp.int32),
        pltpu.VMEM((CH, K), jnp.float32),
        pltpu.VMEM((SP, 128), jnp.float32),
        pltpu.VMEM((K, D_H), jnp.float32),
        pltpu.VMEM_SHARED((N, D_H), jnp.float32),
    ],
)


def _spmm_body(d, tbl_h, src2_h, dst2_h, al2_h, agg_h,
               src2_v, dst2_v, al2_v, r0, acc_sh):
    c = lax.axis_index("c")
    sid = lax.axis_index("s")
    wid = c * NS + sid
    rb = wid * NBK
    nrows = _nrows(wid)

    _zero_acc(r0, acc_sh, sid, d // L)
    plsc.subcore_barrier()

    def chunk(t, carry):
        cb = rb + t * CH
        pltpu.sync_copy(src2_h.at[pl.ds(cb, CH)], src2_v)
        pltpu.sync_copy(dst2_h.at[pl.ds(cb, CH)], dst2_v)
        pltpu.sync_copy(al2_h.at[pl.ds(cb, CH)], al2_v)
        nrem = jnp.clip(nrows - t * CH, 0, CH)
        _pipeline_chunk(tbl_h, acc_sh, src2_v, dst2_v, al2_v, r0, d, nrem)
        return carry

    lax.fori_loop(0, NBK // CH, chunk, 0)
    plsc.subcore_barrier()
    _copy_out(acc_sh, agg_h, c, sid)


def _make_spmm(d):
    return pl.kernel(
        functools.partial(_spmm_body, d),
        out_type=jax.ShapeDtypeStruct((NC, N, d), jnp.float32),
        mesh=_MESH,
        compiler_params=_SC_PARAMS,
        scratch_types=[
            pltpu.VMEM((CH, K), jnp.int32),
            pltpu.VMEM((CH, K), jnp.int32),
            pltpu.VMEM((CH, K), jnp.float32),
            pltpu.VMEM((K, d), jnp.float32),
            pltpu.VMEM_SHARED((N, d), jnp.float32),
        ],
    )


_spmm2 = _make_spmm(D_H)
_spmm3 = _make_spmm(D_OUT)


def _scomb_body(s2_ref, s_ref):
    s_ref[...] = s2_ref[0] + s2_ref[1]


def _scomb(s2):
    return pl.pallas_call(
        _scomb_body,
        grid=(1,),
        in_specs=[pl.BlockSpec((NC, SP, 128), lambda i: (0, 0, 0))],
        out_specs=pl.BlockSpec((SP, 128), lambda i: (0, 0)),
        out_shape=jax.ShapeDtypeStruct((SP, 128), jnp.float32),
    )(s2)


def _mid1_body(agg_ref, b_ref, w_ref, g_ref):
    h = jnp.maximum(agg_ref[0] + agg_ref[1] + b_ref[...], 0.0)
    g_ref[...] = jnp.dot(h, w_ref[...], preferred_element_type=jnp.float32)


def _mid1(agg, b, w):
    return pl.pallas_call(
        _mid1_body,
        grid=(N // _PB,),
        in_specs=[
            pl.BlockSpec((NC, _PB, D_H), lambda i: (0, i, 0)),
            pl.BlockSpec((1, D_H), lambda i: (0, 0)),
            pl.BlockSpec((D_H, D_H), lambda i: (0, 0)),
        ],
        out_specs=pl.BlockSpec((_PB, D_H), lambda i: (i, 0)),
        out_shape=jax.ShapeDtypeStruct((N, D_H), jnp.float32),
    )(agg, b, w)


def _mid2_body(agg_ref, b_ref, w_ref, q_ref):
    h = agg_ref[0] + agg_ref[1] + b_ref[...]
    q_ref[...] = jnp.dot(h, w_ref[...], preferred_element_type=jnp.float32)


def _mid2(agg, b, w):
    return pl.pallas_call(
        _mid2_body,
        grid=(N // _PB,),
        in_specs=[
            pl.BlockSpec((NC, _PB, D_H), lambda i: (0, i, 0)),
            pl.BlockSpec((1, D_H), lambda i: (0, 0)),
            pl.BlockSpec((D_H, D_OUT), lambda i: (0, 0)),
        ],
        out_specs=pl.BlockSpec((_PB, D_OUT), lambda i: (i, 0)),
        out_shape=jax.ShapeDtypeStruct((N, D_OUT), jnp.float32),
    )(agg, b, w)


def _final_body(agg_ref, b_ref, out_ref):
    out_ref[...] = agg_ref[0] + agg_ref[1] + b_ref[...]


def _final(agg, b):
    return pl.pallas_call(
        _final_body,
        grid=(N // _PB,),
        in_specs=[
            pl.BlockSpec((NC, _PB, D_OUT), lambda i: (0, i, 0)),
            pl.BlockSpec((1, D_OUT), lambda i: (0, 0)),
        ],
        out_specs=pl.BlockSpec((_PB, D_OUT), lambda i: (i, 0)),
        out_shape=jax.ShapeDtypeStruct((N, D_OUT), jnp.float32),
    )(agg, b)


def kernel(inputs, edge_index, W_gat, attn_l, attn_r, b_gat, W1, b1, W2, b2):
    src = edge_index[0]
    dst = edge_index[1]
    pad = ((0, EROWS_P - EROWS), (0, 0))
    src2 = jnp.pad(src.reshape(EROWS, K), pad)
    dst2 = jnp.pad(dst.reshape(EROWS, K), pad)
    feat, el2, er2, cvec = _prep(inputs, W_gat,
                                 attn_l.reshape(D_H, 1), attn_r.reshape(D_H, 1))
    el = el2.reshape(N)
    er = er2.reshape(N)
    ex2, s2 = _edge(el, er, src2, dst2, cvec)
    s = _scomb(s2)
    agg1, alpha2 = _spmm1(feat, src2, dst2, ex2, s)
    g = _mid1(agg1, b_gat.reshape(1, D_H), W1)
    agg2 = _spmm2(g, src2, dst2, alpha2)
    q = _mid2(agg2, b1.reshape(1, D_H), W2)
    agg3 = _spmm3(q, src2, dst2, alpha2)
    return _final(agg3, b2.reshape(1, D_OUT))


# speedup vs baseline: 2.6459x; 1.3749x over previous
import functools

import jax
import jax.numpy as jnp
from jax import lax
from jax.experimental import pallas as pl
from jax.experimental.pallas import tpu as pltpu
from jax.experimental.pallas import tpu_sc as plsc

N = 10000
E = 320000
D_IN = 128
D_H = 128
D_OUT = 64
NEG = 0.2

NC, NS, L = 2, 16, 16
NW = NC * NS
K = 128
EROWS = E // K
NBK = 80
EROWS_P = NW * NBK
SP = 80
RPT = 624
_ZOFF = (0, 80, 160, 240, 320, 400, 480, 560)

_MESH = plsc.VectorSubcoreMesh(
    core_axis_name="c", subcore_axis_name="s", num_cores=NC, num_subcores=NS)
_SC_PARAMS = pltpu.CompilerParams(
    use_tc_tiling_on_sc=False, needs_layout_passes=False)


def _nrows(wid):
    return jnp.minimum(EROWS - NBK * wid, NBK)


def _zero_rows(ref, nrows, ncol16):
    z = jnp.zeros((L,), jnp.float32)

    def body(j, carry):
        for cc in range(ncol16):
            ref[j, pl.ds(cc * L, L)] = z
        return carry

    lax.fori_loop(0, nrows, body, 0)


def _prep_body(x_ref, w_ref, al_ref, ar_ref, feat_ref, el_ref, er_ref, cv_ref,
               m_ref):
    i = pl.program_id(0)
    f = jnp.dot(x_ref[...], w_ref[...], preferred_element_type=jnp.float32)
    feat_ref[...] = f
    el = jnp.dot(f, al_ref[...], preferred_element_type=jnp.float32)
    er = jnp.dot(f, ar_ref[...], preferred_element_type=jnp.float32)
    el_ref[...] = el
    er_ref[...] = er

    @pl.when(i == 0)
    def _():
        m_ref[0] = jnp.float32(-1e30)
        m_ref[1] = jnp.float32(-1e30)

    m_ref[0] = jnp.maximum(m_ref[0], jnp.max(el))
    m_ref[1] = jnp.maximum(m_ref[1], jnp.max(er))
    z = m_ref[0] + m_ref[1]
    c = jnp.where(z >= 0, z, NEG * z)
    cv_ref[...] = jnp.full((8, 128), c, jnp.float32)


_PB = 2000


def _prep(x, w, al, ar):
    return pl.pallas_call(
        _prep_body,
        grid=(N // _PB,),
        in_specs=[
            pl.BlockSpec((_PB, D_IN), lambda i: (i, 0)),
            pl.BlockSpec((D_IN, D_H), lambda i: (0, 0)),
            pl.BlockSpec((D_H, 1), lambda i: (0, 0)),
            pl.BlockSpec((D_H, 1), lambda i: (0, 0)),
        ],
        out_specs=[
            pl.BlockSpec((_PB, D_H), lambda i: (i, 0)),
            pl.BlockSpec((_PB, 1), lambda i: (i, 0)),
            pl.BlockSpec((_PB, 1), lambda i: (i, 0)),
            pl.BlockSpec((8, 128), lambda i: (0, 0)),
        ],
        out_shape=[
            jax.ShapeDtypeStruct((N, D_H), jnp.float32),
            jax.ShapeDtypeStruct((N, 1), jnp.float32),
            jax.ShapeDtypeStruct((N, 1), jnp.float32),
            jax.ShapeDtypeStruct((8, 128), jnp.float32),
        ],
        scratch_shapes=[pltpu.SMEM((2,), jnp.float32)],
    )(x, w, al, ar)


def _edge_body(el_h, er_h, src_h, dst_h, cv_h, ex_h, s_h,
               el_v, er_v, src_v, dst_v, ex_v, s_v, cv_v, idx_v, s_sh):
    c = lax.axis_index("c")
    sid = lax.axis_index("s")
    wid = c * NS + sid
    rb = wid * NBK
    pltpu.sync_copy(el_h, el_v)
    pltpu.sync_copy(er_h, er_v)
    pltpu.sync_copy(src_h.at[pl.ds(rb, NBK)], src_v)
    pltpu.sync_copy(dst_h.at[pl.ds(rb, NBK)], dst_v)
    pltpu.sync_copy(cv_h.at[0], cv_v)

    _zero_rows(s_v, SP, 8)

    def ibody(j, carry):
        idx_v[pl.ds(j * L, L)] = lax.iota(jnp.int32, L) + j * L
        return carry

    lax.fori_loop(0, SP // L, ibody, 0)

    @pl.when(sid == 0)
    def _():
        pltpu.sync_copy(s_v, s_sh)

    plsc.subcore_barrier()

    cv = cv_v[pl.ds(0, L)]

    def ebody(r, carry):
        for g in range(K // L):
            sl = pl.ds(g * L, L)
            sv = src_v[r, sl]
            dv = dst_v[r, sl]
            elg = plsc.load_gather(el_v, [sv])
            erg = plsc.load_gather(er_v, [dv])
            z = elg + erg
            e = jnp.where(z >= 0, z, NEG * z)
            ex = jnp.exp(e - cv)
            ex_v[r, sl] = ex
            hi = lax.shift_right_logical(dv, 7)
            lo = lax.bitwise_and(dv, 127)
            plsc.addupdate_scatter(s_v, [hi, lo], ex)
        return carry

    nr = _nrows(wid)
    lax.fori_loop(0, nr, ebody, 0)
    zz = jnp.zeros((L,), jnp.float32)

    def zpad(r, carry):
        for g in range(K // L):
            ex_v[r, pl.ds(g * L, L)] = zz
        return carry

    lax.fori_loop(nr, NBK, zpad, 0)
    pltpu.sync_copy(ex_v, ex_h.at[pl.ds(rb, NBK)])
    pltpu.sync_copy(s_v, s_sh.at[idx_v], add=True)
    plsc.subcore_barrier()

    @pl.when(sid < SP // 8)
    def _():
        pltpu.sync_copy(s_sh.at[pl.ds(sid * 8, 8), :],
                        s_h.at[c, pl.ds(sid * 8, 8), :])


_edge = pl.kernel(
    _edge_body,
    out_type=[
        jax.ShapeDtypeStruct((EROWS_P, K), jnp.float32),
        jax.ShapeDtypeStruct((NC, SP, 128), jnp.float32),
    ],
    mesh=_MESH,
    compiler_params=_SC_PARAMS,
    scratch_types=[
        pltpu.VMEM((N,), jnp.float32),
        pltpu.VMEM((N,), jnp.float32),
        pltpu.VMEM((NBK, K), jnp.int32),
        pltpu.VMEM((NBK, K), jnp.int32),
        pltpu.VMEM((NBK, K), jnp.float32),
        pltpu.VMEM((SP, 128), jnp.float32),
        pltpu.VMEM((128,), jnp.float32),
        pltpu.VMEM((SP,), jnp.int32),
        pltpu.VMEM_SHARED((SP, 128), jnp.float32),
    ],
)


def _copy_out(acc_sh, agg_h, c, sid):
    rowbase = sid * RPT
    pltpu.sync_copy(acc_sh.at[pl.ds(rowbase, RPT), :],
                    agg_h.at[c, pl.ds(rowbase, RPT), :])

    @pl.when(sid == NS - 1)
    def _():
        pltpu.sync_copy(acc_sh.at[pl.ds(NS * RPT, N - NS * RPT), :],
                        agg_h.at[c, pl.ds(NS * RPT, N - NS * RPT), :])


def _zero_acc(rows_v, acc_sh, sid, ncol16):
    _zero_rows(rows_v, NBK, ncol16)
    rowbase = sid * RPT
    for off in _ZOFF:
        pltpu.sync_copy(rows_v.at[pl.ds(0, NBK), :],
                        acc_sh.at[pl.ds(rowbase + off, NBK), :])


CH = 16


def _scale_rows(al2_v, b, rv, d):
    def sg(g, carry):
        av16 = al2_v[b, pl.ds(g * L, L)]
        for j in range(L):
            av = jnp.broadcast_to(av16[j], (L,))
            r = g * L + j
            for cc in range(d // L):
                sl = pl.ds(cc * L, L)
                rv[r, sl] = rv[r, sl] * av
        return carry

    lax.fori_loop(0, K // L, sg, 0)


def _pipeline_chunk(tbl_h, acc_sh, src2_v, dst2_v, al2_v, r0, r1, g0, g1,
                    d, nrem):
    npair = nrem // 2

    def gstart(b, rv, sem):
        pltpu.async_copy(tbl_h.at[src2_v.at[b]], rv, sem)

    def gwait(b, rv, sem):
        pltpu.make_async_copy(tbl_h.at[src2_v.at[b]], rv, sem).wait()

    @pl.when(npair > 0)
    def _():
        gstart(0, r0, g0)

        def pair(u, carry):
            b0 = 2 * u
            b1 = b0 + 1
            gwait(b0, r0, g0)
            gstart(b1, r1, g1)
            _scale_rows(al2_v, b0, r0, d)
            pltpu.sync_copy(r0, acc_sh.at[dst2_v.at[b0]], add=True)
            gwait(b1, r1, g1)

            @pl.when(u < npair - 1)
            def _():
                gstart(b0 + 2, r0, g0)

            _scale_rows(al2_v, b1, r1, d)
            pltpu.sync_copy(r1, acc_sh.at[dst2_v.at[b1]], add=True)
            return carry

        lax.fori_loop(0, npair, pair, 0)


def _spmm1_body(feat_h, src2_h, dst2_h, ex2_h, s_h, agg_h, al2_h,
                src2_v, dst2_v, al2_v, s_v, r0, r1, g0, g1, acc_sh):
    c = lax.axis_index("c")
    sid = lax.axis_index("s")
    wid = c * NS + sid
    rb = wid * NBK
    pltpu.sync_copy(s_h, s_v)
    nrows = _nrows(wid)

    _zero_acc(r0, acc_sh, sid, D_H // L)
    plsc.subcore_barrier()

    def chunk(t, carry):
        cb = rb + t * CH
        pltpu.sync_copy(src2_h.at[pl.ds(cb, CH)], src2_v)
        pltpu.sync_copy(dst2_h.at[pl.ds(cb, CH)], dst2_v)
        pltpu.sync_copy(ex2_h.at[pl.ds(cb, CH)], al2_v)
        nrem = jnp.clip(nrows - t * CH, 0, CH)

        def albody(b, carry2):
            for g in range(K // L):
                sl = pl.ds(g * L, L)
                dv = dst2_v[b, sl]
                hi = lax.shift_right_logical(dv, 7)
                lo = lax.bitwise_and(dv, 127)
                sg = plsc.load_gather(s_v, [hi, lo])
                al2_v[b, sl] = al2_v[b, sl] / (sg + 1e-9)
            return carry2

        lax.fori_loop(0, nrem, albody, 0)
        pltpu.sync_copy(al2_v, al2_h.at[pl.ds(cb, CH)])
        _pipeline_chunk(feat_h, acc_sh, src2_v, dst2_v, al2_v, r0, r1, g0, g1,
                        D_H, nrem)
        return carry

    lax.fori_loop(0, NBK // CH, chunk, 0)
    plsc.subcore_barrier()
    _copy_out(acc_sh, agg_h, c, sid)


_spmm1 = pl.kernel(
    _spmm1_body,
    out_type=[
        jax.ShapeDtypeStruct((NC, N, D_H), jnp.float32),
        jax.ShapeDtypeStruct((EROWS_P, K), jnp.float32),
    ],
    mesh=_MESH,
    compiler_params=_SC_PARAMS,
    scratch_types=[
        pltpu.VMEM((CH, K), jnp.int32),
        pltpu.VMEM((CH, K), jnp.int32),
        pltpu.VMEM((CH, K), jnp.float32),
        pltpu.VMEM((SP, 128), jnp.float32),
        pltpu.VMEM((K, D_H), jnp.float32),
        pltpu.VMEM((K, D_H), jnp.float32),
        pltpu.SemaphoreType.DMA,
        pltpu.SemaphoreType.DMA,
        pltpu.VMEM_SHARED((N, D_H), jnp.float32),
    ],
)


def _spmm_body(d, tbl_h, src2_h, dst2_h, al2_h, agg_h,
               src2_v, dst2_v, al2_v, r0, r1, g0, g1, acc_sh):
    c = lax.axis_index("c")
    sid = lax.axis_index("s")
    wid = c * NS + sid
    rb = wid * NBK
    nrows = _nrows(wid)

    _zero_acc(r0, acc_sh, sid, d // L)
    plsc.subcore_barrier()

    def chunk(t, carry):
        cb = rb + t * CH
        pltpu.sync_copy(src2_h.at[pl.ds(cb, CH)], src2_v)
        pltpu.sync_copy(dst2_h.at[pl.ds(cb, CH)], dst2_v)
        pltpu.sync_copy(al2_h.at[pl.ds(cb, CH)], al2_v)
        nrem = jnp.clip(nrows - t * CH, 0, CH)
        _pipeline_chunk(tbl_h, acc_sh, src2_v, dst2_v, al2_v, r0, r1, g0, g1,
                        d, nrem)
        return carry

    lax.fori_loop(0, NBK // CH, chunk, 0)
    plsc.subcore_barrier()
    _copy_out(acc_sh, agg_h, c, sid)


def _make_spmm(d):
    return pl.kernel(
        functools.partial(_spmm_body, d),
        out_type=jax.ShapeDtypeStruct((NC, N, d), jnp.float32),
        mesh=_MESH,
        compiler_params=_SC_PARAMS,
        scratch_types=[
            pltpu.VMEM((CH, K), jnp.int32),
            pltpu.VMEM((CH, K), jnp.int32),
            pltpu.VMEM((CH, K), jnp.float32),
            pltpu.VMEM((K, d), jnp.float32),
            pltpu.VMEM((K, d), jnp.float32),
            pltpu.SemaphoreType.DMA,
            pltpu.SemaphoreType.DMA,
            pltpu.VMEM_SHARED((N, d), jnp.float32),
        ],
    )


_spmm2 = _make_spmm(D_H)
_spmm3 = _make_spmm(D_OUT)


def _scomb_body(s2_ref, s_ref):
    s_ref[...] = s2_ref[0] + s2_ref[1]


def _scomb(s2):
    return pl.pallas_call(
        _scomb_body,
        grid=(1,),
        in_specs=[pl.BlockSpec((NC, SP, 128), lambda i: (0, 0, 0))],
        out_specs=pl.BlockSpec((SP, 128), lambda i: (0, 0)),
        out_shape=jax.ShapeDtypeStruct((SP, 128), jnp.float32),
    )(s2)


def _mid1_body(agg_ref, b_ref, w_ref, g_ref):
    h = jnp.maximum(agg_ref[0] + agg_ref[1] + b_ref[...], 0.0)
    g_ref[...] = jnp.dot(h, w_ref[...], preferred_element_type=jnp.float32)


def _mid1(agg, b, w):
    return pl.pallas_call(
        _mid1_body,
        grid=(N // _PB,),
        in_specs=[
            pl.BlockSpec((NC, _PB, D_H), lambda i: (0, i, 0)),
            pl.BlockSpec((1, D_H), lambda i: (0, 0)),
            pl.BlockSpec((D_H, D_H), lambda i: (0, 0)),
        ],
        out_specs=pl.BlockSpec((_PB, D_H), lambda i: (i, 0)),
        out_shape=jax.ShapeDtypeStruct((N, D_H), jnp.float32),
    )(agg, b, w)


def _mid2_body(agg_ref, b_ref, w_ref, q_ref):
    h = agg_ref[0] + agg_ref[1] + b_ref[...]
    q_ref[...] = jnp.dot(h, w_ref[...], preferred_element_type=jnp.float32)


def _mid2(agg, b, w):
    return pl.pallas_call(
        _mid2_body,
        grid=(N // _PB,),
        in_specs=[
            pl.BlockSpec((NC, _PB, D_H), lambda i: (0, i, 0)),
            pl.BlockSpec((1, D_H), lambda i: (0, 0)),
            pl.BlockSpec((D_H, D_OUT), lambda i: (0, 0)),
        ],
        out_specs=pl.BlockSpec((_PB, D_OUT), lambda i: (i, 0)),
        out_shape=jax.ShapeDtypeStruct((N, D_OUT), jnp.float32),
    )(agg, b, w)


def _final_body(agg_ref, b_ref, out_ref):
    out_ref[...] = agg_ref[0] + agg_ref[1] + b_ref[...]


def _final(agg, b):
    return pl.pallas_call(
        _final_body,
        grid=(N // _PB,),
        in_specs=[
            pl.BlockSpec((NC, _PB, D_OUT), lambda i: (0, i, 0)),
            pl.BlockSpec((1, D_OUT), lambda i: (0, 0)),
        ],
        out_specs=pl.BlockSpec((_PB, D_OUT), lambda i: (i, 0)),
        out_shape=jax.ShapeDtypeStruct((N, D_OUT), jnp.float32),
    )(agg, b)


def kernel(inputs, edge_index, W_gat, attn_l, attn_r, b_gat, W1, b1, W2, b2):
    src = edge_index[0]
    dst = edge_index[1]
    pad = ((0, EROWS_P - EROWS), (0, 0))
    src2 = jnp.pad(src.reshape(EROWS, K), pad)
    dst2 = jnp.pad(dst.reshape(EROWS, K), pad)
    feat, el2, er2, cvec = _prep(inputs, W_gat,
                                 attn_l.reshape(D_H, 1), attn_r.reshape(D_H, 1))
    el = el2.reshape(N)
    er = er2.reshape(N)
    ex2, s2 = _edge(el, er, src2, dst2, cvec)
    s = _scomb(s2)
    agg1, alpha2 = _spmm1(feat, src2, dst2, ex2, s)
    g = _mid1(agg1, b_gat.reshape(1, D_H), W1)
    agg2 = _spmm2(g, src2, dst2, alpha2)
    q = _mid2(agg2, b1.reshape(1, D_H), W2)
    agg3 = _spmm3(q, src2, dst2, alpha2)
    return _final(agg3, b2.reshape(1, D_OUT))
